# Initial kernel scaffold; baseline (speedup 1.0000x reference)
import functools

import jax
import jax.numpy as jnp
from jax import lax
from jax.experimental import pallas as pl
from jax.experimental.pallas import tpu as pltpu
from jax.experimental.pallas import tpu_sc as plsc

_B = 4096
_N = 2 * _B
_DIM = 64
_MAXLEN = 50
_CHARSET = 128
_MARGIN = 1.0
_NC = 2
_NS = 16
_NW = _NC * _NS
_TPW = _N // _NW
_CHUNK = 128
_LANES = 16


def _sc_body(heads_hbm, rels_hbm, tails_hbm, ent_hbm, rel_hbm, attr_hbm,
             ent_out, rel_out, cnt_out,
             hidx, ridx, tidx, ent_v, rel_v, chars_v, cnt_v, sem_a, sem_b):
  w = lax.axis_index("s") * _NC + lax.axis_index("c")
  row0 = w * (_TPW // _CHUNK)
  base = w * _TPW

  pltpu.sync_copy(heads_hbm.at[pl.ds(row0, 2)], hidx)
  pltpu.sync_copy(rels_hbm.at[pl.ds(row0, 2)], ridx)
  pltpu.sync_copy(tails_hbm.at[pl.ds(row0, 2)], tidx)

  ch0 = pltpu.async_copy(attr_hbm.at[tidx.at[0]], chars_v.at[pl.ds(0, _CHUNK)], sem_b)
  ch1 = pltpu.async_copy(attr_hbm.at[tidx.at[1]], chars_v.at[pl.ds(_CHUNK, _CHUNK)], sem_b)
  e0 = pltpu.async_copy(ent_hbm.at[hidx.at[0]], ent_v.at[pl.ds(0, _CHUNK)], sem_a)
  e1 = pltpu.async_copy(ent_hbm.at[hidx.at[1]], ent_v.at[pl.ds(_CHUNK, _CHUNK)], sem_a)
  r0 = pltpu.async_copy(rel_hbm.at[ridx.at[0]], rel_v.at[pl.ds(0, _CHUNK)], sem_a)
  r1 = pltpu.async_copy(rel_hbm.at[ridx.at[1]], rel_v.at[pl.ds(_CHUNK, _CHUNK)], sem_a)

  zero16 = jnp.zeros((_LANES,), jnp.float32)

  def zbody(i, c):
    cnt_v[i >> 3, pl.ds((i & 7) * _LANES, _LANES)] = zero16
    return c

  lax.fori_loop(0, _TPW * (_CHARSET // _LANES), zbody, 0)

  ch0.wait()
  ch1.wait()

  iota = lax.iota(jnp.int32, _LANES)
  ones = jnp.ones((_LANES,), jnp.float32)
  tail_mask = iota + 3 * _LANES < _MAXLEN

  def tbody(t, c):
    trow = jnp.full((_LANES,), t, jnp.int32)
    for k in range(3):
      ch = plsc.load_gather(chars_v, [trow, iota + k * _LANES])
      plsc.addupdate_scatter(cnt_v, [trow, ch], ones)
    ch = plsc.load_gather(chars_v, [trow, iota + 3 * _LANES], mask=tail_mask)
    plsc.addupdate_scatter(cnt_v, [trow, ch], ones, mask=tail_mask)
    return c

  lax.fori_loop(0, _TPW, tbody, 0)

  e0.wait()
  e1.wait()
  r0.wait()
  r1.wait()

  pltpu.sync_copy(ent_v, ent_out.at[pl.ds(base, _TPW)])
  pltpu.sync_copy(rel_v, rel_out.at[pl.ds(base, _TPW)])
  pltpu.sync_copy(cnt_v, cnt_out.at[pl.ds(base, _TPW)])


_sc_gather = functools.partial(
    pl.kernel,
    out_type=(
        jax.ShapeDtypeStruct((_N, _DIM), jnp.float32),
        jax.ShapeDtypeStruct((_N, _DIM), jnp.float32),
        jax.ShapeDtypeStruct((_N, _CHARSET), jnp.float32),
    ),
    mesh=plsc.VectorSubcoreMesh(core_axis_name="c", subcore_axis_name="s"),
    scratch_types=(
        pltpu.VMEM((2, _CHUNK), jnp.int32),
        pltpu.VMEM((2, _CHUNK), jnp.int32),
        pltpu.VMEM((2, _CHUNK), jnp.int32),
        pltpu.VMEM((_TPW, _DIM), jnp.float32),
        pltpu.VMEM((_TPW, _DIM), jnp.float32),
        pltpu.VMEM((_TPW, _MAXLEN), jnp.int32),
        pltpu.VMEM((_TPW, _CHARSET), jnp.float32),
        pltpu.SemaphoreType.DMA,
        pltpu.SemaphoreType.DMA,
    ),
)(_sc_body)


def _tc_body(ent_ref, rel_ref, cnt_ref, tbl_ref, out_ref):
  e = ent_ref[...]
  inv = lax.rsqrt(jnp.sum(e * e, axis=1, keepdims=True))
  attr = jnp.dot(cnt_ref[...], tbl_ref[...], preferred_element_type=jnp.float32)
  diff = e * inv + rel_ref[...] - attr
  s = jnp.sum(jnp.abs(diff), axis=1, keepdims=True)
  out_ref[...] = jnp.maximum(s[:_B] - s[_B:] + _MARGIN, 0.0)


_tc_dense = pl.pallas_call(
    _tc_body,
    out_shape=jax.ShapeDtypeStruct((_B, 1), jnp.float32),
)


def kernel(positive_triplets, negative_triplets, entities_emb_a,
           rel_embeddings, char_embeddings, attr_lookup_table_a):
  trip = jnp.concatenate([positive_triplets, negative_triplets], axis=0)
  trip = trip.astype(jnp.int32)
  heads = trip[:, 0].reshape(_N // _CHUNK, _CHUNK)
  rels = trip[:, 1].reshape(_N // _CHUNK, _CHUNK)
  tails = trip[:, 2].reshape(_N // _CHUNK, _CHUNK)
  ent_rows, rel_rows, counts = _sc_gather(
      heads, rels, tails, entities_emb_a, rel_embeddings,
      attr_lookup_table_a.astype(jnp.int32))
  out = _tc_dense(ent_rows, rel_rows, counts, char_embeddings)
  return out.reshape(_B)


# speedup vs baseline: 7.3797x; 7.3797x over previous
import functools

import jax
import jax.numpy as jnp
from jax import lax
from jax.experimental import pallas as pl
from jax.experimental.pallas import tpu as pltpu
from jax.experimental.pallas import tpu_sc as plsc

_B = 4096
_N = 2 * _B
_DIM = 64
_MAXLEN = 50
_CHARSET = 128
_MARGIN = 1.0
_NC = 2
_NS = 16
_NW = _NC * _NS
_TPW = _N // _NW
_CHUNK = 128
_L = 16
_W = 128
_G = 512
_NP = _B // _G
_RSZ = _G * _CHARSET + 2 * _CHARSET
_ZPW = _RSZ // _NS
_DUMP = _G * _CHARSET
_BIG = 1 << 24


def _sc_body(heads_hbm, rels_hbm, tails_hbm, ent_hbm, rel_hbm, attr_hbm,
             ent_out, rel_out, cnt_out,
             hidx, ridx, tidx, ent_v, rel_v, chars_v, idxf_v, idxp_v,
             ones_v, zero_v, cnt_sh, sem_a, sem_b):
  s_id = lax.axis_index("s")
  c_id = lax.axis_index("c")
  w = c_id * _NS + s_id
  row0 = w * (_TPW // _CHUNK)
  base = w * _TPW

  pltpu.sync_copy(heads_hbm.at[pl.ds(row0, 2)], hidx)
  pltpu.sync_copy(rels_hbm.at[pl.ds(row0, 2)], ridx)
  pltpu.sync_copy(tails_hbm.at[pl.ds(row0, 2)], tidx)

  ch0 = pltpu.async_copy(attr_hbm.at[tidx.at[0]], chars_v.at[pl.ds(0, _CHUNK)], sem_b)
  ch1 = pltpu.async_copy(attr_hbm.at[tidx.at[1]], chars_v.at[pl.ds(_CHUNK, _CHUNK)], sem_b)
  e0 = pltpu.async_copy(ent_hbm.at[hidx.at[0]], ent_v.at[pl.ds(0, _CHUNK)], sem_a)
  e1 = pltpu.async_copy(ent_hbm.at[hidx.at[1]], ent_v.at[pl.ds(_CHUNK, _CHUNK)], sem_a)
  r0 = pltpu.async_copy(rel_hbm.at[ridx.at[0]], rel_v.at[pl.ds(0, _CHUNK)], sem_a)
  r1 = pltpu.async_copy(rel_hbm.at[ridx.at[1]], rel_v.at[pl.ds(_CHUNK, _CHUNK)], sem_a)

  one16 = jnp.ones((_L,), jnp.float32)
  zero16 = jnp.zeros((_L,), jnp.float32)

  def _init(i, c):
    zero_v[pl.ds(i * _L, _L)] = zero16
    return c

  lax.fori_loop(0, _ZPW // _L, _init, 0)
  for j in range(8):
    ones_v[pl.ds(j * _L, _L)] = one16

  ch0.wait()
  ch1.wait()

  iota = lax.iota(jnp.int32, _L)
  tailm = iota < (_MAXLEN - 3 * _L)
  big = jnp.full((_L,), _BIG, jnp.int32)

  def _bld(q, c):
    fb = jnp.full((_L,), (s_id * _TPW + q) * _MAXLEN, jnp.int32)
    r = q >> 1
    cb = (q & 1) * 64
    for k in range(4):
      col = iota + k * _L
      ch = chars_v[q, pl.ds(k * _L, _L)]
      ch = jnp.minimum(jnp.maximum(ch, 0), _CHARSET - 1)
      t = (fb + col) & (_B - 1)
      slot = (t << 7) + ch
      if k == 3:
        slot = jnp.where(tailm, slot, big)
      idxf_v[r, pl.ds(cb + k * _L, _L)] = slot
    return c

  lax.fori_loop(0, _TPW, _bld, 0)

  dump = jnp.full((_L,), _DUMP, jnp.int32) + iota

  for p in range(_NP):
    lo = p * _G * _CHARSET
    hi = lo + _G * _CHARSET
    for j in range(2):
      pltpu.sync_copy(zero_v.at[pl.ds(j * (_ZPW // 2), _ZPW // 2)],
                      cnt_sh.at[pl.ds(s_id * _ZPW + j * (_ZPW // 2),
                                      _ZPW // 2)])
    plsc.subcore_barrier()

    def _scat(r, c):
      q0s = (s_id * _TPW + 2 * r) * _MAXLEN
      st0 = q0s & (_B - 1)
      en0 = (q0s + 2 * _MAXLEN - 1) & (_B - 1)
      pl_lo = p * _G
      hit0 = jnp.logical_and(st0 >= pl_lo, st0 < pl_lo + _G)
      hit1 = jnp.logical_and(en0 >= pl_lo, en0 < pl_lo + _G)

      @pl.when(jnp.logical_or(hit0, hit1))
      def _():
        for k in range(8):
          sl = idxf_v[r, pl.ds(k * _L, _L)]
          inr = jnp.logical_and(sl >= lo, sl < hi)
          idxp_v[0, pl.ds(k * _L, _L)] = jnp.where(inr, sl - lo, dump)
        pltpu.sync_copy(ones_v, cnt_sh.at[idxp_v.at[0]], add=True)

      return c

    lax.fori_loop(0, _TPW // 2, _scat, 0)
    plsc.subcore_barrier()
    pltpu.sync_copy(
        cnt_sh.at[pl.ds(s_id * (_G // _NS) * _CHARSET, (_G // _NS) * _CHARSET)],
        cnt_out.at[pl.ds((c_id * _B + p * _G + s_id * (_G // _NS)) * _CHARSET,
                         (_G // _NS) * _CHARSET)])
    plsc.subcore_barrier()

  e0.wait()
  e1.wait()
  r0.wait()
  r1.wait()

  pltpu.sync_copy(ent_v, ent_out.at[pl.ds(base, _TPW)])
  pltpu.sync_copy(rel_v, rel_out.at[pl.ds(base, _TPW)])


@functools.cache
def _build_sc_gather():
  return functools.partial(
      pl.kernel,
      out_type=(
          jax.ShapeDtypeStruct((_N, _W), jnp.float32),
          jax.ShapeDtypeStruct((_N, _W), jnp.float32),
          jax.ShapeDtypeStruct((_N * _CHARSET,), jnp.float32),
      ),
      mesh=plsc.VectorSubcoreMesh(core_axis_name="c", subcore_axis_name="s",
                                  num_cores=_NC, num_subcores=_NS),
      compiler_params=pltpu.CompilerParams(use_tc_tiling_on_sc=False),
      scratch_types=(
          pltpu.VMEM((2, _CHUNK), jnp.int32),
          pltpu.VMEM((2, _CHUNK), jnp.int32),
          pltpu.VMEM((2, _CHUNK), jnp.int32),
          pltpu.VMEM((_TPW, _W), jnp.float32),
          pltpu.VMEM((_TPW, _W), jnp.float32),
          pltpu.VMEM((_TPW, _W), jnp.int32),
          pltpu.VMEM((_TPW // 2, _CHUNK), jnp.int32),
          pltpu.VMEM((1, _CHUNK), jnp.int32),
          pltpu.VMEM((_CHUNK,), jnp.float32),
          pltpu.VMEM((_ZPW,), jnp.float32),
          pltpu.VMEM_SHARED((_NS * _ZPW,), jnp.float32),
          pltpu.SemaphoreType.DMA,
          pltpu.SemaphoreType.DMA,
      ),
  )(_sc_body)


def _tc_body(ent_ref, rel_ref, cnt_ref, tbl_ref, hpar_ref, rpar_ref, out_ref):
  el = ent_ref[...]
  rl = rel_ref[...]
  e = jnp.where(hpar_ref[...] == 0, el[:, :_DIM], el[:, _DIM:])
  r = jnp.where(rpar_ref[...] == 0, rl[:, :_DIM], rl[:, _DIM:])
  inv = lax.rsqrt(jnp.sum(e * e, axis=1, keepdims=True))
  attr = jnp.dot(cnt_ref[...], tbl_ref[...], preferred_element_type=jnp.float32)
  diff = e * inv + r - attr
  s = jnp.sum(jnp.abs(diff), axis=1, keepdims=True)
  out_ref[...] = jnp.maximum(s[:_B] - s[_B:] + _MARGIN, 0.0)


_tc_dense = pl.pallas_call(
    _tc_body,
    out_shape=jax.ShapeDtypeStruct((_B, 1), jnp.float32),
)


def kernel(positive_triplets, negative_triplets, entities_emb_a,
           rel_embeddings, char_embeddings, attr_lookup_table_a):
  trip = jnp.concatenate([positive_triplets, negative_triplets], axis=0)
  trip = trip.astype(jnp.int32)
  heads = trip[:, 0]
  rels = trip[:, 1]
  tails = trip[:, 2].reshape(_N // _CHUNK, _CHUNK)
  hpair = (heads >> 1).reshape(_N // _CHUNK, _CHUNK)
  rpair = (rels >> 1).reshape(_N // _CHUNK, _CHUNK)
  ent2 = entities_emb_a.reshape(-1, _W)
  rel2 = rel_embeddings.reshape(-1, _W)
  attr_pad = jnp.pad(attr_lookup_table_a.astype(jnp.int32),
                     ((0, 0), (0, _W - _MAXLEN)))
  ent_land, rel_land, counts = _build_sc_gather()(
      hpair, rpair, tails, ent2, rel2, attr_pad)
  out = _tc_dense(ent_land, rel_land, counts.reshape(_N, _CHARSET),
                  char_embeddings, (heads & 1).reshape(_N, 1),
                  (rels & 1).reshape(_N, 1))
  return out.reshape(_B)
